# Initial kernel scaffold; baseline (speedup 1.0000x reference)
import jax
import jax.numpy as jnp
from jax.experimental import pallas as pl


def kernel(x, adj, W1, W2, W3):
    raise NotImplementedError("write your pallas kernel here")


# speedup vs baseline: 1.5089x; 1.5089x over previous
import functools

import jax
import jax.numpy as jnp
from jax.experimental import pallas as pl
from jax.experimental.pallas import tpu as pltpu


def _p1_kernel(x_ref, adj_ref, w1_ref, h_ref, s1_ref):
    @pl.when(pl.program_id(0) == 0)
    def _():
        s1_ref[...] = jnp.dot(x_ref[...], w1_ref[...],
                              preferred_element_type=jnp.float32)

    h_ref[...] = jnp.maximum(
        jnp.dot(adj_ref[...], s1_ref[...], preferred_element_type=jnp.float32),
        0.0)


def _p2_kernel(h_ref, adj_ref, w23_ref, out_ref, b_ref, acc_ref, *, nb, z, nz):
    i = pl.program_id(0)

    @pl.when(i == 0)
    def _():
        b_ref[...] = jnp.dot(h_ref[...], w23_ref[...],
                             preferred_element_type=jnp.float32)
        acc_ref[0] = 0.0
        acc_ref[1] = 0.0

    l = jnp.dot(adj_ref[...], b_ref[...], preferred_element_type=jnp.float32)
    acc_ref[0] += jnp.sum(jnp.exp(l[:, :z]) - 1.0)
    acc_ref[1] += jnp.sum(l[:, z:])

    @pl.when(i == nb - 1)
    def _():
        out_ref[0] = acc_ref[1] / nz
        out_ref[1] = jnp.log(1.0 + acc_ref[0] / nz)


def _pick_bm(n):
    for bm in (400, 200, 80, 40, 16, 8):
        if n % bm == 0:
            return bm
    return n


@jax.jit
def kernel(x, adj, W1, W2, W3):
    n, d = x.shape
    h_dim = W1.shape[1]
    z = W2.shape[1]
    bm = _pick_bm(n)
    nb = n // bm

    hidden1 = pl.pallas_call(
        _p1_kernel,
        grid=(nb,),
        in_specs=[
            pl.BlockSpec((n, d), lambda i: (0, 0)),
            pl.BlockSpec((bm, n), lambda i: (i, 0)),
            pl.BlockSpec((d, h_dim), lambda i: (0, 0)),
        ],
        out_specs=pl.BlockSpec((bm, h_dim), lambda i: (i, 0)),
        out_shape=jax.ShapeDtypeStruct((n, h_dim), jnp.float32),
        scratch_shapes=[pltpu.VMEM((n, h_dim), jnp.float32)],
    )(x, adj, W1)

    w23 = jnp.concatenate([W3, W2], axis=1)

    out = pl.pallas_call(
        functools.partial(_p2_kernel, nb=nb, z=z, nz=float(n * z)),
        grid=(nb,),
        in_specs=[
            pl.BlockSpec((n, h_dim), lambda i: (0, 0)),
            pl.BlockSpec((bm, n), lambda i: (i, 0)),
            pl.BlockSpec((h_dim, 2 * z), lambda i: (0, 0)),
        ],
        out_specs=pl.BlockSpec(memory_space=pltpu.SMEM),
        out_shape=jax.ShapeDtypeStruct((2,), jnp.float32),
        scratch_shapes=[
            pltpu.VMEM((n, 2 * z), jnp.float32),
            pltpu.SMEM((2,), jnp.float32),
        ],
    )(hidden1, adj, w23)

    return (out[0].reshape(()), out[1].reshape(()))
